# Initial kernel scaffold; baseline (speedup 1.0000x reference)
import jax
import jax.numpy as jnp
from jax.experimental import pallas as pl


def kernel(x, emb, W, b):
    raise NotImplementedError("write your pallas kernel here")


# speedup vs baseline: 15.4954x; 15.4954x over previous
import functools

import jax
import jax.numpy as jnp
from jax import lax
from jax.experimental import pallas as pl
from jax.experimental.pallas import tpu as pltpu
from jax.experimental.pallas import tpu_sc as plsc

VOCAB = 1000000
EMBED_DIM = 16
TAG_SIZE = 2
BATCH = 16384
HIST = 50

_INFO = plsc.get_sparse_core_info()
_NC = _INFO.num_cores
_NS = _INFO.num_subcores
_NW = _NC * _NS
_N = BATCH * HIST
_PER_W = _N // _NW
_CHUNK = 2560
_NCHUNK = _PER_W // _CHUNK
_SUB = 128
_NSUB = _CHUNK // _SUB
_GROUPS = _CHUNK // 16


def _body(x_hbm, emb_hbm, w_hbm, b_hbm, out_hbm, idx_v, rows_v, out_v,
          w_v, b_v, sem):
    wid = lax.axis_index("s") * _NC + lax.axis_index("c")
    base = wid * _PER_W

    pltpu.sync_copy(w_hbm, w_v)
    pltpu.sync_copy(b_hbm, b_v)
    w_rows = [w_v[t, :] for t in range(TAG_SIZE)]
    w_s = [[w_rows[t][d] for d in range(EMBED_DIM)] for t in range(TAG_SIZE)]
    b_vec = b_v[:]
    b_s = [b_vec[t] for t in range(TAG_SIZE)]

    lanes = lax.iota(jnp.int32, 16)

    for c in range(_NCHUNK):
        cbase = base + c * _CHUNK
        pltpu.sync_copy(x_hbm.at[pl.ds(cbase, _CHUNK)], idx_v)

        handles = []
        for j in range(_NSUB):
            h = pltpu.async_copy(
                emb_hbm.at[idx_v.at[pl.ds(j * _SUB, _SUB)]],
                rows_v.at[pl.ds(j * _SUB, _SUB), :],
                sem,
            )
            handles.append(h)
        for h in handles:
            h.wait()

        def compute(g, carry):
            row_ids = g * 16 + lanes
            acc0 = jnp.full((16,), b_s[0], dtype=jnp.float32)
            acc1 = jnp.full((16,), b_s[1], dtype=jnp.float32)
            for d in range(EMBED_DIM):
                col_ids = jnp.full((16,), d, dtype=jnp.int32)
                col = plsc.load_gather(rows_v, [row_ids, col_ids])
                acc0 = acc0 + col * w_s[0][d]
                acc1 = acc1 + col * w_s[1][d]
            out_ids = g * 32 + lanes * 2
            plsc.store_scatter(out_v, [out_ids], acc0)
            plsc.store_scatter(out_v, [out_ids + 1], acc1)
            return carry

        lax.fori_loop(0, _GROUPS, compute, 0)

        pltpu.sync_copy(out_v,
                        out_hbm.at[pl.ds(cbase * TAG_SIZE, _CHUNK * TAG_SIZE)])


@jax.jit
def _run(x_flat, emb, W, b_pad):
    mesh = plsc.VectorSubcoreMesh(core_axis_name="c", subcore_axis_name="s")
    return pl.kernel(
        _body,
        out_type=jax.ShapeDtypeStruct((_N * TAG_SIZE,), jnp.float32),
        mesh=mesh,
        scratch_types=[
            pltpu.VMEM((_CHUNK,), jnp.int32),
            pltpu.VMEM((_CHUNK, EMBED_DIM), jnp.float32),
            pltpu.VMEM((_CHUNK * TAG_SIZE,), jnp.float32),
            pltpu.VMEM((TAG_SIZE, EMBED_DIM), jnp.float32),
            pltpu.VMEM((16,), jnp.float32),
            pltpu.SemaphoreType.DMA,
        ],
        compiler_params=pltpu.CompilerParams(
            needs_layout_passes=False, use_tc_tiling_on_sc=False),
    )(x_flat, emb, W, b_pad)


def kernel(x, emb, W, b):
    x_flat = x.reshape(-1).astype(jnp.int32)
    b_pad = jnp.pad(b.astype(jnp.float32), (0, 16 - TAG_SIZE))
    out = _run(x_flat, emb, W, b_pad)
    return out.reshape(BATCH, HIST, TAG_SIZE)


# speedup vs baseline: 30.4320x; 1.9639x over previous
import jax
import jax.numpy as jnp
from jax import lax
from jax.experimental import pallas as pl
from jax.experimental.pallas import tpu as pltpu
from jax.experimental.pallas import tpu_sc as plsc

VOCAB = 1000000
EMBED_DIM = 16
TAG_SIZE = 2
BATCH = 16384
HIST = 50

_INFO = plsc.get_sparse_core_info()
_NC = _INFO.num_cores
_NS = _INFO.num_subcores
_NW = _NC * _NS
_N = BATCH * HIST
_BPW = BATCH // _NW
_PER_W = _BPW * HIST
_CHUNK = 1280
_NCHUNK = _PER_W // _CHUNK
_SUB = 128
_NSUB = _CHUNK // _SUB
_GROUPS = _CHUNK // 16


def _body(x_hbm, emb_hbm, w_hbm, b_hbm, out_hbm, idx_v, rows0_v, rows1_v,
          out_v, w_v, b_v, sem0, sem1):
    wid = lax.axis_index("s") * _NC + lax.axis_index("c")
    base = wid * _PER_W

    pltpu.sync_copy(w_hbm, w_v)
    pltpu.sync_copy(b_hbm, b_v)
    w_rows = [w_v[t, :] for t in range(TAG_SIZE)]
    w_s = [[w_rows[t][d] for d in range(EMBED_DIM)] for t in range(TAG_SIZE)]
    b_vec = b_v[:]
    b_s = [b_vec[t] for t in range(TAG_SIZE)]

    lanes = lax.iota(jnp.int32, 16)

    pltpu.sync_copy(x_hbm.at[pl.ds(base, _PER_W)], idx_v)

    bufs = (rows0_v, rows1_v)
    sems = (sem0, sem1)

    def fire(c):
        handles = []
        for j in range(_NSUB):
            handles.append(pltpu.async_copy(
                emb_hbm.at[idx_v.at[pl.ds(c * _CHUNK + j * _SUB, _SUB)]],
                bufs[c % 2].at[pl.ds(j * _SUB, _SUB), :],
                sems[c % 2],
            ))
        return handles

    pending = fire(0)
    for c in range(_NCHUNK):
        for h in pending:
            h.wait()
        if c + 1 < _NCHUNK:
            pending = fire(c + 1)
        rows = bufs[c % 2]

        def compute(g, carry):
            row_ids = g * 16 + lanes
            i_local = c * _CHUNK + row_ids
            b_off = i_local // HIST
            l_pos = i_local - b_off * HIST
            t0 = jnp.zeros((16,), dtype=jnp.int32)
            t1 = jnp.full((16,), 1, dtype=jnp.int32)
            acc0 = jnp.full((16,), b_s[0], dtype=jnp.float32)
            acc1 = jnp.full((16,), b_s[1], dtype=jnp.float32)
            for d in range(EMBED_DIM):
                col_ids = jnp.full((16,), d, dtype=jnp.int32)
                col = plsc.load_gather(rows, [row_ids, col_ids])
                acc0 = acc0 + col * w_s[0][d]
                acc1 = acc1 + col * w_s[1][d]
            plsc.store_scatter(out_v, [l_pos, t0, b_off], acc0)
            plsc.store_scatter(out_v, [l_pos, t1, b_off], acc1)
            return carry

        lax.fori_loop(0, _GROUPS, compute, 0)

    pltpu.sync_copy(out_v, out_hbm.at[:, :, pl.ds(wid * _BPW, _BPW)])


@jax.jit
def _run(x_flat, emb, W, b_pad):
    mesh = plsc.VectorSubcoreMesh(core_axis_name="c", subcore_axis_name="s")
    return pl.kernel(
        _body,
        out_type=jax.ShapeDtypeStruct((HIST, TAG_SIZE, BATCH), jnp.float32),
        mesh=mesh,
        scratch_types=[
            pltpu.VMEM((_PER_W,), jnp.int32),
            pltpu.VMEM((_CHUNK, EMBED_DIM), jnp.float32),
            pltpu.VMEM((_CHUNK, EMBED_DIM), jnp.float32),
            pltpu.VMEM((HIST, TAG_SIZE, _BPW), jnp.float32),
            pltpu.VMEM((TAG_SIZE, EMBED_DIM), jnp.float32),
            pltpu.VMEM((16,), jnp.float32),
            pltpu.SemaphoreType.DMA,
            pltpu.SemaphoreType.DMA,
        ],
        compiler_params=pltpu.CompilerParams(
            needs_layout_passes=False, use_tc_tiling_on_sc=False),
    )(x_flat, emb, W, b_pad)


def kernel(x, emb, W, b):
    x_flat = x.reshape(-1).astype(jnp.int32)
    b_pad = jnp.pad(b.astype(jnp.float32), (0, 16 - TAG_SIZE))
    out_t = _run(x_flat, emb, W, b_pad)
    return jnp.transpose(out_t, (2, 0, 1))
